# Initial kernel scaffold; baseline (speedup 1.0000x reference)
import jax
import jax.numpy as jnp
from jax.experimental import pallas as pl


def kernel(x, edge_index, W1, b1, W2, b2, Wc, bc):
    raise NotImplementedError("write your pallas kernel here")


# speedup vs baseline: 12.5333x; 12.5333x over previous
import functools

import jax
import jax.numpy as jnp
from jax import lax
from jax.experimental import pallas as pl
from jax.experimental.pallas import tpu as pltpu
from jax.experimental.pallas import tpu_sc as plsc

N = 10000
E = 320000
F = 128
NPAD = 10240
RPT = NPAD // 16
B = 80
EROWS = E // B

_MESH = dict(core_axis_name="c", subcore_axis_name="s", num_cores=2,
             num_subcores=16)


def _zero_rows(ref, nrows, ncols):
    zeros = jnp.zeros((16,), jnp.float32)

    def body(i, carry):
        for j in range(ncols // 16):
            ref[i, pl.ds(j * 16, 16)] = zeros
        return carry

    lax.fori_loop(0, nrows, body, 0)


def _zero_vec(ref, n):
    zeros = jnp.zeros((16,), jnp.float32)

    def body(i, carry):
        ref[pl.ds(i * 16, 16)] = zeros
        return carry

    lax.fori_loop(0, n // 16, body, 0)


def _sc_deg_kernel():
    return pl.kernel(
        _sc_deg_body,
        out_type=jax.ShapeDtypeStruct((2, NPAD), jnp.float32),
        mesh=plsc.VectorSubcoreMesh(**_MESH),
        scratch_types=[
            pltpu.VMEM((EROWS // 16, B), jnp.int32),
            pltpu.VMEM((B,), jnp.float32),
            pltpu.VMEM((RPT,), jnp.float32),
            pltpu.VMEM_SHARED((NPAD,), jnp.float32),
            pltpu.SemaphoreType.DMA,
        ],
    )


def _sc_deg_body(edges_hbm, out_hbm, idxv, ones, zv, acc, sem):
    c = lax.axis_index("c")
    s = lax.axis_index("s")
    base = s * RPT
    nrows = EROWS // 16

    o = jnp.ones((16,), jnp.float32)
    for j in range(B // 16):
        ones[pl.ds(j * 16, 16)] = o
    _zero_vec(zv, RPT)
    pltpu.sync_copy(zv, acc.at[pl.ds(base, RPT)])
    pltpu.sync_copy(edges_hbm.at[c, s], idxv)
    plsc.subcore_barrier()

    def fire(i, carry):
        pltpu.async_copy(ones, acc.at[idxv.at[i]], sem, add=True)
        return carry

    lax.fori_loop(0, nrows, fire, 0)

    def drain(i, carry):
        pltpu.make_async_copy(ones, acc.at[idxv.at[i]], sem).wait()
        return carry

    lax.fori_loop(0, nrows, drain, 0)
    plsc.subcore_barrier()
    pltpu.sync_copy(acc.at[pl.ds(base, RPT)], out_hbm.at[c, pl.ds(base, RPT)])


HF = F // 2


def _sc_agg_body(src_hbm, dst_hbm, zcat_hbm, out_hbm, sidx, didx, rows0,
                 rows1, acc, sem0, sem1):
    c = lax.axis_index("c")
    s = lax.axis_index("s")
    base = s * RPT
    nb = EROWS // 16

    _zero_rows(rows0, B, HF)

    def zcopy(i, carry):
        pltpu.sync_copy(rows0, acc.at[pl.ds(base + i * B, B)])
        return carry

    lax.fori_loop(0, RPT // B, zcopy, 0)

    pltpu.sync_copy(src_hbm.at[s], sidx)
    pltpu.sync_copy(dst_hbm.at[s], didx)

    off = jnp.full((16,), c * NPAD, jnp.int32)

    def adj(i, carry):
        for j in range(B // 16):
            sidx[i, pl.ds(j * 16, 16)] = sidx[i, pl.ds(j * 16, 16)] + off
        return carry

    lax.fori_loop(0, nb, adj, 0)
    plsc.subcore_barrier()

    pltpu.async_copy(zcat_hbm.at[sidx.at[0]], rows0, sem0)

    def body(i, carry):
        b0 = 2 * i
        pltpu.async_copy(zcat_hbm.at[sidx.at[b0 + 1]], rows1, sem1)
        pltpu.make_async_copy(zcat_hbm.at[sidx.at[b0]], rows0, sem0).wait()
        pltpu.sync_copy(rows0, acc.at[didx.at[b0]], add=True)

        @pl.when(b0 + 2 < nb)
        def _():
            pltpu.async_copy(zcat_hbm.at[sidx.at[b0 + 2]], rows0, sem0)

        pltpu.make_async_copy(zcat_hbm.at[sidx.at[b0 + 1]], rows1,
                              sem1).wait()
        pltpu.sync_copy(rows1, acc.at[didx.at[b0 + 1]], add=True)
        return carry

    lax.fori_loop(0, nb // 2, body, 0)

    plsc.subcore_barrier()
    pltpu.sync_copy(acc.at[pl.ds(base, RPT)],
                    out_hbm.at[c, pl.ds(base, RPT)])


def _sc_agg_kernel():
    return pl.kernel(
        _sc_agg_body,
        out_type=jax.ShapeDtypeStruct((2, NPAD, HF), jnp.float32),
        mesh=plsc.VectorSubcoreMesh(**_MESH),
        scratch_types=[
            pltpu.VMEM((EROWS // 16, B), jnp.int32),
            pltpu.VMEM((EROWS // 16, B), jnp.int32),
            pltpu.VMEM((B, HF), jnp.float32),
            pltpu.VMEM((B, HF), jnp.float32),
            pltpu.VMEM_SHARED((NPAD, HF), jnp.float32),
            pltpu.SemaphoreType.DMA,
            pltpu.SemaphoreType.DMA,
        ],
        compiler_params=pltpu.CompilerParams(use_tc_tiling_on_sc=False),
    )


def _sc_final_body(src_hbm, dst_hbm, vn_hbm, nd_hbm, c0_hbm, y_hbm, sidx,
                   didx, vals, workv, ndv, c0v, acc, sem):
    c = lax.axis_index("c")
    s = lax.axis_index("s")
    base = s * RPT
    nrows = EROWS // 16

    @pl.when(c == 0)
    def _():
        _zero_vec(workv, RPT)
        pltpu.sync_copy(workv, acc.at[pl.ds(base, RPT)])
        pltpu.sync_copy(src_hbm.at[s], sidx)
        pltpu.sync_copy(dst_hbm.at[s], didx)
        plsc.subcore_barrier()

        def fire_g(i, carry):
            pltpu.async_copy(vn_hbm.at[sidx.at[i]], vals.at[i], sem)
            return carry

        lax.fori_loop(0, nrows, fire_g, 0)

        def drain_g(i, carry):
            pltpu.make_async_copy(vn_hbm.at[sidx.at[i]], vals.at[i],
                                  sem).wait()
            return carry

        lax.fori_loop(0, nrows, drain_g, 0)

        def fire_s(i, carry):
            pltpu.async_copy(vals.at[i], acc.at[didx.at[i]], sem, add=True)
            return carry

        lax.fori_loop(0, nrows, fire_s, 0)

        def drain_s(i, carry):
            pltpu.make_async_copy(vals.at[i], acc.at[didx.at[i]], sem).wait()
            return carry

        lax.fori_loop(0, nrows, drain_s, 0)
        plsc.subcore_barrier()
        pltpu.sync_copy(acc.at[pl.ds(base, RPT)], workv)
        pltpu.sync_copy(nd_hbm.at[pl.ds(base, RPT)], ndv)
        pltpu.sync_copy(c0_hbm.at[pl.ds(0, 16)], c0v)
        c016 = c0v[pl.ds(0, 16)]

        def fin(i, carry):
            a = workv[pl.ds(i * 16, 16)]
            d = ndv[pl.ds(i * 16, 16)]
            workv[pl.ds(i * 16, 16)] = a * d + c016
            return carry

        lax.fori_loop(0, RPT // 16, fin, 0)
        pltpu.sync_copy(workv, y_hbm.at[pl.ds(base, RPT)])


def _sc_final_kernel():
    return pl.kernel(
        _sc_final_body,
        out_type=jax.ShapeDtypeStruct((NPAD,), jnp.float32),
        mesh=plsc.VectorSubcoreMesh(**_MESH),
        scratch_types=[
            pltpu.VMEM((EROWS // 16, B), jnp.int32),
            pltpu.VMEM((EROWS // 16, B), jnp.int32),
            pltpu.VMEM((EROWS // 16, B), jnp.float32),
            pltpu.VMEM((RPT,), jnp.float32),
            pltpu.VMEM((RPT,), jnp.float32),
            pltpu.VMEM((16,), jnp.float32),
            pltpu.VMEM_SHARED((NPAD,), jnp.float32),
            pltpu.SemaphoreType.DMA,
        ],
    )


def _tc1_body(x_ref, w_ref, do_ref, di_ref, zn_ref, ns_ref, nd_ref):
    do = do_ref[...]
    di = di_ref[...]
    ns = jnp.where(do > 0, lax.rsqrt(jnp.maximum(do, 1.0)), 0.0)
    nd = jnp.where(di > 0, lax.rsqrt(jnp.maximum(di, 1.0)), 0.0)
    z = jnp.dot(x_ref[...], w_ref[0], preferred_element_type=jnp.float32)
    zn_ref[...] = (z * ns)[None]
    ns_ref[...] = ns
    nd_ref[...] = nd


def _tc1(x_pad, W1, dego, degi):
    g = 8
    rb = NPAD // g
    return pl.pallas_call(
        _tc1_body,
        grid=(2, g),
        in_specs=[
            pl.BlockSpec((rb, F), lambda j, i: (i, 0)),
            pl.BlockSpec((1, F, HF), lambda j, i: (j, 0, 0)),
            pl.BlockSpec((rb, 1), lambda j, i: (i, 0)),
            pl.BlockSpec((rb, 1), lambda j, i: (i, 0)),
        ],
        out_specs=[
            pl.BlockSpec((1, rb, HF), lambda j, i: (j, i, 0)),
            pl.BlockSpec((rb, 1), lambda j, i: (i, 0)),
            pl.BlockSpec((rb, 1), lambda j, i: (i, 0)),
        ],
        out_shape=[
            jax.ShapeDtypeStruct((2, NPAD, HF), jnp.float32),
            jax.ShapeDtypeStruct((NPAD, 1), jnp.float32),
            jax.ShapeDtypeStruct((NPAD, 1), jnp.float32),
        ],
    )(x_pad, jnp.stack([W1[:, :HF], W1[:, HF:]]), dego, degi)


def _tc2_body(p_ref, nd_ref, ns_ref, b1_ref, w2_ref, wc_ref, b2_ref, bc_ref,
              vn_ref, c0_ref):
    agg = jnp.concatenate([p_ref[0], p_ref[1]], axis=1)
    h1 = jnp.maximum(agg * nd_ref[...] + b1_ref[...], 0.0)
    w2c = jnp.dot(w2_ref[...], wc_ref[...], preferred_element_type=jnp.float32)
    v = jnp.dot(h1, w2c, preferred_element_type=jnp.float32)
    vn_ref[...] = v * ns_ref[...]
    c0 = jnp.dot(b2_ref[...], wc_ref[...],
                 preferred_element_type=jnp.float32) + bc_ref[...]
    c0_ref[...] = jnp.broadcast_to(c0, c0_ref.shape)


def _tc2(P, nd_col, ns_col, b1r, W2, Wc, b2r, bcr):
    g = 8
    rb = NPAD // g
    return pl.pallas_call(
        _tc2_body,
        grid=(g,),
        in_specs=[
            pl.BlockSpec((2, rb, HF), lambda i: (0, i, 0)),
            pl.BlockSpec((rb, 1), lambda i: (i, 0)),
            pl.BlockSpec((rb, 1), lambda i: (i, 0)),
            pl.BlockSpec((1, F), lambda i: (0, 0)),
            pl.BlockSpec((F, F), lambda i: (0, 0)),
            pl.BlockSpec((F, 1), lambda i: (0, 0)),
            pl.BlockSpec((1, F), lambda i: (0, 0)),
            pl.BlockSpec((1, 1), lambda i: (0, 0)),
        ],
        out_specs=[
            pl.BlockSpec((rb, 1), lambda i: (i, 0)),
            pl.BlockSpec((1, F), lambda i: (0, 0)),
        ],
        out_shape=[
            jax.ShapeDtypeStruct((NPAD, 1), jnp.float32),
            jax.ShapeDtypeStruct((1, F), jnp.float32),
        ],
    )(P, nd_col, ns_col, b1r, W2, Wc, b2r, bcr)


def kernel(x, edge_index, W1, b1, W2, b2, Wc, bc):
    edges16 = edge_index.reshape(2, 16, EROWS // 16, B)
    src16 = edge_index[0].reshape(16, EROWS // 16, B)
    dst16 = edge_index[1].reshape(16, EROWS // 16, B)
    x_pad = jnp.pad(x, ((0, NPAD - N), (0, 0)))

    degs = _sc_deg_kernel()(edges16)
    dego = degs[0].reshape(NPAD, 1)
    degi = degs[1].reshape(NPAD, 1)

    zsplit, ns_col, nd_col = _tc1(x_pad, W1, dego, degi)
    zcat = zsplit.reshape(2 * NPAD, HF)

    P = _sc_agg_kernel()(src16, dst16, zcat)

    vn_col, c0 = _tc2(P, nd_col, ns_col, b1.reshape(1, F), W2, Wc,
                      b2.reshape(1, F), bc.reshape(1, 1))

    y_pad = _sc_final_kernel()(src16, dst16, vn_col.reshape(NPAD),
                               nd_col.reshape(NPAD), c0.reshape(F))
    return y_pad[:N].reshape(N, 1)


# speedup vs baseline: 14.7459x; 1.1765x over previous
import functools

import jax
import jax.numpy as jnp
from jax import lax
from jax.experimental import pallas as pl
from jax.experimental.pallas import tpu as pltpu
from jax.experimental.pallas import tpu_sc as plsc

N = 10000
E = 320000
F = 128
NPAD = 10240
RPT = NPAD // 16
B = 80
EROWS = E // B

_MESH = dict(core_axis_name="c", subcore_axis_name="s", num_cores=2,
             num_subcores=16)


def _zero_rows(ref, nrows, ncols):
    zeros = jnp.zeros((16,), jnp.float32)

    def body(i, carry):
        for j in range(ncols // 16):
            ref[i, pl.ds(j * 16, 16)] = zeros
        return carry

    lax.fori_loop(0, nrows, body, 0)


def _zero_vec(ref, n):
    zeros = jnp.zeros((16,), jnp.float32)

    def body(i, carry):
        ref[pl.ds(i * 16, 16)] = zeros
        return carry

    lax.fori_loop(0, n // 16, body, 0)


def _sc_deg_kernel():
    return pl.kernel(
        _sc_deg_body,
        out_type=jax.ShapeDtypeStruct((2, NPAD), jnp.float32),
        mesh=plsc.VectorSubcoreMesh(**_MESH),
        scratch_types=[
            pltpu.VMEM((EROWS // 16, B), jnp.int32),
            pltpu.VMEM((B,), jnp.float32),
            pltpu.VMEM((RPT,), jnp.float32),
            pltpu.VMEM_SHARED((NPAD,), jnp.float32),
            pltpu.SemaphoreType.DMA,
        ],
    )


def _sc_deg_body(edges_hbm, out_hbm, idxv, ones, zv, acc, sem):
    c = lax.axis_index("c")
    s = lax.axis_index("s")
    base = s * RPT
    nrows = EROWS // 16

    o = jnp.ones((16,), jnp.float32)
    for j in range(B // 16):
        ones[pl.ds(j * 16, 16)] = o
    _zero_vec(zv, RPT)
    pltpu.sync_copy(zv, acc.at[pl.ds(base, RPT)])
    pltpu.sync_copy(edges_hbm.at[c, s], idxv)
    plsc.subcore_barrier()

    def fire(i, carry):
        pltpu.async_copy(ones, acc.at[idxv.at[i]], sem, add=True)
        return carry

    lax.fori_loop(0, nrows, fire, 0)

    def drain(i, carry):
        pltpu.make_async_copy(ones, acc.at[idxv.at[i]], sem).wait()
        return carry

    lax.fori_loop(0, nrows, drain, 0)
    plsc.subcore_barrier()
    pltpu.sync_copy(acc.at[pl.ds(base, RPT)], out_hbm.at[c, pl.ds(base, RPT)])


HF = F // 2


def _sc_agg_body(src_hbm, dst_hbm, zcat_hbm, out_hbm, sidx, didx, rows0,
                 rows1, rows2, rows3, acc, gs0, gs1, gs2, gs3, ss0, ss1,
                 ss2, ss3):
    c = lax.axis_index("c")
    s = lax.axis_index("s")
    base = s * RPT
    nb = EROWS // 16

    _zero_rows(rows0, B, HF)

    def zcopy(i, carry):
        pltpu.sync_copy(rows0, acc.at[pl.ds(base + i * B, B)])
        return carry

    lax.fori_loop(0, RPT // B, zcopy, 0)

    pltpu.sync_copy(src_hbm.at[s], sidx)
    pltpu.sync_copy(dst_hbm.at[s], didx)

    off = jnp.full((16,), c * NPAD, jnp.int32)

    def adj(i, carry):
        for j in range(B // 16):
            sidx[i, pl.ds(j * 16, 16)] = sidx[i, pl.ds(j * 16, 16)] + off
        return carry

    lax.fori_loop(0, nb, adj, 0)
    plsc.subcore_barrier()

    rows = (rows0, rows1, rows2, rows3)
    gsem = (gs0, gs1, gs2, gs3)
    ssem = (ss0, ss1, ss2, ss3)

    def gath(b, j):
        return pltpu.async_copy(zcat_hbm.at[sidx.at[b]], rows[j], gsem[j])

    def gath_w(b, j):
        pltpu.make_async_copy(zcat_hbm.at[sidx.at[b]], rows[j],
                              gsem[j]).wait()

    def scat(b, j):
        return pltpu.async_copy(rows[j], acc.at[didx.at[b]], ssem[j],
                                add=True)

    def scat_w(b, j):
        pltpu.make_async_copy(rows[j], acc.at[didx.at[b]], ssem[j]).wait()

    gath(0, 0)
    gath(1, 1)

    def body(i, carry):
        for k in range(4):
            m = 4 * i + k

            @pl.when(m >= 2)
            def _():
                scat_w(m - 2, (k + 2) % 4)

            gath(m + 2, (k + 2) % 4)
            gath_w(m, k)
            scat(m, k)
        return carry

    lax.fori_loop(0, (nb - 2) // 4, body, 0)
    scat_w(nb - 4, (nb - 4) % 4)
    gath_w(nb - 2, (nb - 2) % 4)
    scat(nb - 2, (nb - 2) % 4)
    scat_w(nb - 3, (nb - 3) % 4)
    gath_w(nb - 1, (nb - 1) % 4)
    scat(nb - 1, (nb - 1) % 4)
    scat_w(nb - 2, (nb - 2) % 4)
    scat_w(nb - 1, (nb - 1) % 4)

    plsc.subcore_barrier()
    pltpu.sync_copy(acc.at[pl.ds(base, RPT)],
                    out_hbm.at[c, pl.ds(base, RPT)])


def _sc_agg_kernel():
    return pl.kernel(
        _sc_agg_body,
        out_type=jax.ShapeDtypeStruct((2, NPAD, HF), jnp.float32),
        mesh=plsc.VectorSubcoreMesh(**_MESH),
        scratch_types=[
            pltpu.VMEM((EROWS // 16, B), jnp.int32),
            pltpu.VMEM((EROWS // 16, B), jnp.int32),
            pltpu.VMEM((B, HF), jnp.float32),
            pltpu.VMEM((B, HF), jnp.float32),
            pltpu.VMEM((B, HF), jnp.float32),
            pltpu.VMEM((B, HF), jnp.float32),
            pltpu.VMEM_SHARED((NPAD, HF), jnp.float32),
            pltpu.SemaphoreType.DMA,
            pltpu.SemaphoreType.DMA,
            pltpu.SemaphoreType.DMA,
            pltpu.SemaphoreType.DMA,
            pltpu.SemaphoreType.DMA,
            pltpu.SemaphoreType.DMA,
            pltpu.SemaphoreType.DMA,
            pltpu.SemaphoreType.DMA,
        ],
        compiler_params=pltpu.CompilerParams(use_tc_tiling_on_sc=False),
    )


def _sc_final_body(src_hbm, dst_hbm, vn_hbm, nd_hbm, c0_hbm, y_hbm, sidx,
                   didx, vals, workv, ndv, c0v, acc, sem):
    c = lax.axis_index("c")
    s = lax.axis_index("s")
    base = s * RPT
    nrows = EROWS // 16

    @pl.when(c == 0)
    def _():
        _zero_vec(workv, RPT)
        pltpu.sync_copy(workv, acc.at[pl.ds(base, RPT)])
        pltpu.sync_copy(src_hbm.at[s], sidx)
        pltpu.sync_copy(dst_hbm.at[s], didx)
        plsc.subcore_barrier()

        def fire_g(i, carry):
            pltpu.async_copy(vn_hbm.at[sidx.at[i]], vals.at[i], sem)
            return carry

        lax.fori_loop(0, nrows, fire_g, 0)

        def drain_g(i, carry):
            pltpu.make_async_copy(vn_hbm.at[sidx.at[i]], vals.at[i],
                                  sem).wait()
            return carry

        lax.fori_loop(0, nrows, drain_g, 0)

        def fire_s(i, carry):
            pltpu.async_copy(vals.at[i], acc.at[didx.at[i]], sem, add=True)
            return carry

        lax.fori_loop(0, nrows, fire_s, 0)

        def drain_s(i, carry):
            pltpu.make_async_copy(vals.at[i], acc.at[didx.at[i]], sem).wait()
            return carry

        lax.fori_loop(0, nrows, drain_s, 0)
        plsc.subcore_barrier()
        pltpu.sync_copy(acc.at[pl.ds(base, RPT)], workv)
        pltpu.sync_copy(nd_hbm.at[pl.ds(base, RPT)], ndv)
        pltpu.sync_copy(c0_hbm.at[pl.ds(0, 16)], c0v)
        c016 = c0v[pl.ds(0, 16)]

        def fin(i, carry):
            a = workv[pl.ds(i * 16, 16)]
            d = ndv[pl.ds(i * 16, 16)]
            workv[pl.ds(i * 16, 16)] = a * d + c016
            return carry

        lax.fori_loop(0, RPT // 16, fin, 0)
        pltpu.sync_copy(workv, y_hbm.at[pl.ds(base, RPT)])


def _sc_final_kernel():
    return pl.kernel(
        _sc_final_body,
        out_type=jax.ShapeDtypeStruct((NPAD,), jnp.float32),
        mesh=plsc.VectorSubcoreMesh(**_MESH),
        scratch_types=[
            pltpu.VMEM((EROWS // 16, B), jnp.int32),
            pltpu.VMEM((EROWS // 16, B), jnp.int32),
            pltpu.VMEM((EROWS // 16, B), jnp.float32),
            pltpu.VMEM((RPT,), jnp.float32),
            pltpu.VMEM((RPT,), jnp.float32),
            pltpu.VMEM((16,), jnp.float32),
            pltpu.VMEM_SHARED((NPAD,), jnp.float32),
            pltpu.SemaphoreType.DMA,
        ],
    )


def _tc1_body(x_ref, w_ref, do_ref, di_ref, zn_ref, ns_ref, nd_ref):
    do = do_ref[...]
    di = di_ref[...]
    ns = jnp.where(do > 0, lax.rsqrt(jnp.maximum(do, 1.0)), 0.0)
    nd = jnp.where(di > 0, lax.rsqrt(jnp.maximum(di, 1.0)), 0.0)
    z = jnp.dot(x_ref[...], w_ref[0], preferred_element_type=jnp.float32)
    zn_ref[...] = (z * ns)[None]
    ns_ref[...] = ns
    nd_ref[...] = nd


def _tc1(x_pad, W1, dego, degi):
    g = 8
    rb = NPAD // g
    return pl.pallas_call(
        _tc1_body,
        grid=(2, g),
        in_specs=[
            pl.BlockSpec((rb, F), lambda j, i: (i, 0)),
            pl.BlockSpec((1, F, HF), lambda j, i: (j, 0, 0)),
            pl.BlockSpec((rb, 1), lambda j, i: (i, 0)),
            pl.BlockSpec((rb, 1), lambda j, i: (i, 0)),
        ],
        out_specs=[
            pl.BlockSpec((1, rb, HF), lambda j, i: (j, i, 0)),
            pl.BlockSpec((rb, 1), lambda j, i: (i, 0)),
            pl.BlockSpec((rb, 1), lambda j, i: (i, 0)),
        ],
        out_shape=[
            jax.ShapeDtypeStruct((2, NPAD, HF), jnp.float32),
            jax.ShapeDtypeStruct((NPAD, 1), jnp.float32),
            jax.ShapeDtypeStruct((NPAD, 1), jnp.float32),
        ],
    )(x_pad, jnp.stack([W1[:, :HF], W1[:, HF:]]), dego, degi)


def _tc2_body(p_ref, nd_ref, ns_ref, b1_ref, w2_ref, wc_ref, b2_ref, bc_ref,
              vn_ref, c0_ref):
    agg = jnp.concatenate([p_ref[0], p_ref[1]], axis=1)
    h1 = jnp.maximum(agg * nd_ref[...] + b1_ref[...], 0.0)
    w2c = jnp.dot(w2_ref[...], wc_ref[...], preferred_element_type=jnp.float32)
    v = jnp.dot(h1, w2c, preferred_element_type=jnp.float32)
    vn_ref[...] = v * ns_ref[...]
    c0 = jnp.dot(b2_ref[...], wc_ref[...],
                 preferred_element_type=jnp.float32) + bc_ref[...]
    c0_ref[...] = jnp.broadcast_to(c0, c0_ref.shape)


def _tc2(P, nd_col, ns_col, b1r, W2, Wc, b2r, bcr):
    g = 8
    rb = NPAD // g
    return pl.pallas_call(
        _tc2_body,
        grid=(g,),
        in_specs=[
            pl.BlockSpec((2, rb, HF), lambda i: (0, i, 0)),
            pl.BlockSpec((rb, 1), lambda i: (i, 0)),
            pl.BlockSpec((rb, 1), lambda i: (i, 0)),
            pl.BlockSpec((1, F), lambda i: (0, 0)),
            pl.BlockSpec((F, F), lambda i: (0, 0)),
            pl.BlockSpec((F, 1), lambda i: (0, 0)),
            pl.BlockSpec((1, F), lambda i: (0, 0)),
            pl.BlockSpec((1, 1), lambda i: (0, 0)),
        ],
        out_specs=[
            pl.BlockSpec((rb, 1), lambda i: (i, 0)),
            pl.BlockSpec((1, F), lambda i: (0, 0)),
        ],
        out_shape=[
            jax.ShapeDtypeStruct((NPAD, 1), jnp.float32),
            jax.ShapeDtypeStruct((1, F), jnp.float32),
        ],
    )(P, nd_col, ns_col, b1r, W2, Wc, b2r, bcr)


def kernel(x, edge_index, W1, b1, W2, b2, Wc, bc):
    edges16 = edge_index.reshape(2, 16, EROWS // 16, B)
    src16 = edge_index[0].reshape(16, EROWS // 16, B)
    dst16 = edge_index[1].reshape(16, EROWS // 16, B)
    x_pad = jnp.pad(x, ((0, NPAD - N), (0, 0)))

    degs = _sc_deg_kernel()(edges16)
    dego = degs[0].reshape(NPAD, 1)
    degi = degs[1].reshape(NPAD, 1)

    zsplit, ns_col, nd_col = _tc1(x_pad, W1, dego, degi)
    zcat = zsplit.reshape(2 * NPAD, HF)

    P = _sc_agg_kernel()(src16, dst16, zcat)

    vn_col, c0 = _tc2(P, nd_col, ns_col, b1.reshape(1, F), W2, Wc,
                      b2.reshape(1, F), bc.reshape(1, 1))

    y_pad = _sc_final_kernel()(src16, dst16, vn_col.reshape(NPAD),
                               nd_col.reshape(NPAD), c0.reshape(F))
    return y_pad[:N].reshape(N, 1)


# speedup vs baseline: 17.2891x; 1.1725x over previous
import functools

import jax
import jax.numpy as jnp
from jax import lax
from jax.experimental import pallas as pl
from jax.experimental.pallas import tpu as pltpu
from jax.experimental.pallas import tpu_sc as plsc

N = 10000
E = 320000
F = 128
NPAD = 10240
RPT = NPAD // 16
B = 80
EROWS = E // B

_MESH = dict(core_axis_name="c", subcore_axis_name="s", num_cores=2,
             num_subcores=16)


def _zero_rows(ref, nrows, ncols):
    zeros = jnp.zeros((16,), jnp.float32)

    def body(i, carry):
        for j in range(ncols // 16):
            ref[i, pl.ds(j * 16, 16)] = zeros
        return carry

    lax.fori_loop(0, nrows, body, 0)


def _zero_vec(ref, n):
    zeros = jnp.zeros((16,), jnp.float32)

    def body(i, carry):
        ref[pl.ds(i * 16, 16)] = zeros
        return carry

    lax.fori_loop(0, n // 16, body, 0)


def _sc_deg_kernel():
    return pl.kernel(
        _sc_deg_body,
        out_type=jax.ShapeDtypeStruct((2, NPAD), jnp.float32),
        mesh=plsc.VectorSubcoreMesh(**_MESH),
        scratch_types=[
            pltpu.VMEM((EROWS // 16, B), jnp.int32),
            pltpu.VMEM((B,), jnp.float32),
            pltpu.VMEM((RPT,), jnp.float32),
            pltpu.VMEM_SHARED((NPAD,), jnp.float32),
            pltpu.SemaphoreType.DMA,
        ],
    )


def _sc_deg_body(edges_hbm, out_hbm, idxv, ones, zv, acc, sem):
    c = lax.axis_index("c")
    s = lax.axis_index("s")
    base = s * RPT
    nrows = EROWS // 16

    o = jnp.ones((16,), jnp.float32)
    for j in range(B // 16):
        ones[pl.ds(j * 16, 16)] = o
    _zero_vec(zv, RPT)
    pltpu.sync_copy(zv, acc.at[pl.ds(base, RPT)])
    pltpu.sync_copy(edges_hbm.at[c, s], idxv)
    plsc.subcore_barrier()

    def fire(i, carry):
        pltpu.async_copy(ones, acc.at[idxv.at[i]], sem, add=True)
        return carry

    lax.fori_loop(0, nrows, fire, 0)

    def drain(i, carry):
        pltpu.make_async_copy(ones, acc.at[idxv.at[i]], sem).wait()
        return carry

    lax.fori_loop(0, nrows, drain, 0)
    plsc.subcore_barrier()
    pltpu.sync_copy(acc.at[pl.ds(base, RPT)], out_hbm.at[c, pl.ds(base, RPT)])


HF = F // 2


def _sc_agg_body(src_hbm, dst_hbm, zcat_hbm, out_hbm, sidx, didx, rows0,
                 rows1, rows2, rows3, acc, gs0, gs1, gs2, gs3, ss0, ss1,
                 ss2, ss3):
    c = lax.axis_index("c")
    s = lax.axis_index("s")
    base = s * RPT
    nb = EROWS // 16

    _zero_rows(rows0, B, HF)

    def zcopy(i, carry):
        pltpu.sync_copy(rows0, acc.at[pl.ds(base + i * B, B)])
        return carry

    lax.fori_loop(0, RPT // B, zcopy, 0)

    pltpu.sync_copy(src_hbm.at[s], sidx)
    pltpu.sync_copy(dst_hbm.at[s], didx)

    off = jnp.full((16,), c * NPAD, jnp.int32)

    def adj(i, carry):
        for j in range(B // 16):
            sidx[i, pl.ds(j * 16, 16)] = sidx[i, pl.ds(j * 16, 16)] + off
        return carry

    lax.fori_loop(0, nb, adj, 0)
    plsc.subcore_barrier()

    rows = (rows0, rows1, rows2, rows3)
    gsem = (gs0, gs1, gs2, gs3)
    ssem = (ss0, ss1, ss2, ss3)

    def gath(b, j):
        return pltpu.async_copy(zcat_hbm.at[sidx.at[b]], rows[j], gsem[j])

    def gath_w(b, j):
        pltpu.make_async_copy(zcat_hbm.at[sidx.at[b]], rows[j],
                              gsem[j]).wait()

    def scat(b, j):
        return pltpu.async_copy(rows[j], acc.at[didx.at[b]], ssem[j],
                                add=True)

    def scat_w(b, j):
        pltpu.make_async_copy(rows[j], acc.at[didx.at[b]], ssem[j]).wait()

    gath(0, 0)
    gath(1, 1)

    def body(i, carry):
        for k in range(4):
            m = 4 * i + k

            @pl.when(m >= 2)
            def _():
                scat_w(m - 2, (k + 2) % 4)

            gath(m + 2, (k + 2) % 4)
            gath_w(m, k)
            scat(m, k)
        return carry

    lax.fori_loop(0, (nb - 2) // 4, body, 0)
    scat_w(nb - 4, (nb - 4) % 4)
    gath_w(nb - 2, (nb - 2) % 4)
    scat(nb - 2, (nb - 2) % 4)
    scat_w(nb - 3, (nb - 3) % 4)
    gath_w(nb - 1, (nb - 1) % 4)
    scat(nb - 1, (nb - 1) % 4)
    scat_w(nb - 2, (nb - 2) % 4)
    scat_w(nb - 1, (nb - 1) % 4)

    plsc.subcore_barrier()
    pltpu.sync_copy(acc.at[pl.ds(base, RPT)],
                    out_hbm.at[c, pl.ds(base, RPT)])


def _sc_agg_kernel():
    return pl.kernel(
        _sc_agg_body,
        out_type=jax.ShapeDtypeStruct((2, NPAD, HF), jnp.float32),
        mesh=plsc.VectorSubcoreMesh(**_MESH),
        scratch_types=[
            pltpu.VMEM((EROWS // 16, B), jnp.int32),
            pltpu.VMEM((EROWS // 16, B), jnp.int32),
            pltpu.VMEM((B, HF), jnp.float32),
            pltpu.VMEM((B, HF), jnp.float32),
            pltpu.VMEM((B, HF), jnp.float32),
            pltpu.VMEM((B, HF), jnp.float32),
            pltpu.VMEM_SHARED((NPAD, HF), jnp.float32),
            pltpu.SemaphoreType.DMA,
            pltpu.SemaphoreType.DMA,
            pltpu.SemaphoreType.DMA,
            pltpu.SemaphoreType.DMA,
            pltpu.SemaphoreType.DMA,
            pltpu.SemaphoreType.DMA,
            pltpu.SemaphoreType.DMA,
            pltpu.SemaphoreType.DMA,
        ],
        compiler_params=pltpu.CompilerParams(use_tc_tiling_on_sc=False),
    )


def _sc_final_body(src_hbm, dst_hbm, vn_hbm, nd_hbm, c0_hbm, y_hbm, sidx,
                   didx, vals, vns, workv, ndv, c0v, acc, sem):
    c = lax.axis_index("c")
    s = lax.axis_index("s")
    base = s * RPT
    nrows = EROWS // 16

    @pl.when(c == 0)
    def _():
        _zero_vec(workv, RPT)
        pltpu.sync_copy(workv, acc.at[pl.ds(base, RPT)])
        pltpu.sync_copy(src_hbm.at[s], sidx)
        pltpu.sync_copy(dst_hbm.at[s], didx)
        pltpu.sync_copy(vn_hbm.at[pl.ds(base, RPT)],
                        vns.at[pl.ds(base, RPT)])
        plsc.subcore_barrier()

        def fire_g(i, carry):
            pltpu.async_copy(vns.at[sidx.at[i]], vals.at[i], sem)
            return carry

        lax.fori_loop(0, nrows, fire_g, 0)

        def drain_g(i, carry):
            pltpu.make_async_copy(vns.at[sidx.at[i]], vals.at[i],
                                  sem).wait()
            return carry

        lax.fori_loop(0, nrows, drain_g, 0)

        def fire_s(i, carry):
            pltpu.async_copy(vals.at[i], acc.at[didx.at[i]], sem, add=True)
            return carry

        lax.fori_loop(0, nrows, fire_s, 0)

        def drain_s(i, carry):
            pltpu.make_async_copy(vals.at[i], acc.at[didx.at[i]], sem).wait()
            return carry

        lax.fori_loop(0, nrows, drain_s, 0)
        plsc.subcore_barrier()
        pltpu.sync_copy(acc.at[pl.ds(base, RPT)], workv)
        pltpu.sync_copy(nd_hbm.at[pl.ds(base, RPT)], ndv)
        pltpu.sync_copy(c0_hbm.at[pl.ds(0, 16)], c0v)
        c016 = c0v[pl.ds(0, 16)]

        def fin(i, carry):
            a = workv[pl.ds(i * 16, 16)]
            d = ndv[pl.ds(i * 16, 16)]
            workv[pl.ds(i * 16, 16)] = a * d + c016
            return carry

        lax.fori_loop(0, RPT // 16, fin, 0)
        pltpu.sync_copy(workv, y_hbm.at[pl.ds(base, RPT)])


def _sc_final_kernel():
    return pl.kernel(
        _sc_final_body,
        out_type=jax.ShapeDtypeStruct((NPAD,), jnp.float32),
        mesh=plsc.VectorSubcoreMesh(**_MESH),
        scratch_types=[
            pltpu.VMEM((EROWS // 16, B), jnp.int32),
            pltpu.VMEM((EROWS // 16, B), jnp.int32),
            pltpu.VMEM((EROWS // 16, B), jnp.float32),
            pltpu.VMEM_SHARED((NPAD,), jnp.float32),
            pltpu.VMEM((RPT,), jnp.float32),
            pltpu.VMEM((RPT,), jnp.float32),
            pltpu.VMEM((16,), jnp.float32),
            pltpu.VMEM_SHARED((NPAD,), jnp.float32),
            pltpu.SemaphoreType.DMA,
        ],
    )


def _tc1_body(x_ref, w_ref, do_ref, di_ref, zn_ref, ns_ref, nd_ref):
    do = do_ref[...]
    di = di_ref[...]
    ns = jnp.where(do > 0, lax.rsqrt(jnp.maximum(do, 1.0)), 0.0)
    nd = jnp.where(di > 0, lax.rsqrt(jnp.maximum(di, 1.0)), 0.0)
    z = jnp.dot(x_ref[...], w_ref[0], preferred_element_type=jnp.float32)
    zn_ref[...] = (z * ns)[None]
    ns_ref[...] = ns
    nd_ref[...] = nd


def _tc1(x_pad, W1, dego, degi):
    g = 8
    rb = NPAD // g
    return pl.pallas_call(
        _tc1_body,
        grid=(2, g),
        in_specs=[
            pl.BlockSpec((rb, F), lambda j, i: (i, 0)),
            pl.BlockSpec((1, F, HF), lambda j, i: (j, 0, 0)),
            pl.BlockSpec((rb, 1), lambda j, i: (i, 0)),
            pl.BlockSpec((rb, 1), lambda j, i: (i, 0)),
        ],
        out_specs=[
            pl.BlockSpec((1, rb, HF), lambda j, i: (j, i, 0)),
            pl.BlockSpec((rb, 1), lambda j, i: (i, 0)),
            pl.BlockSpec((rb, 1), lambda j, i: (i, 0)),
        ],
        out_shape=[
            jax.ShapeDtypeStruct((2, NPAD, HF), jnp.float32),
            jax.ShapeDtypeStruct((NPAD, 1), jnp.float32),
            jax.ShapeDtypeStruct((NPAD, 1), jnp.float32),
        ],
    )(x_pad, jnp.stack([W1[:, :HF], W1[:, HF:]]), dego, degi)


def _tc2_body(p_ref, nd_ref, ns_ref, b1_ref, w2_ref, wc_ref, b2_ref, bc_ref,
              vn_ref, c0_ref):
    agg = jnp.concatenate([p_ref[0], p_ref[1]], axis=1)
    h1 = jnp.maximum(agg * nd_ref[...] + b1_ref[...], 0.0)
    w2c = jnp.dot(w2_ref[...], wc_ref[...], preferred_element_type=jnp.float32)
    v = jnp.dot(h1, w2c, preferred_element_type=jnp.float32)
    vn_ref[...] = v * ns_ref[...]
    c0 = jnp.dot(b2_ref[...], wc_ref[...],
                 preferred_element_type=jnp.float32) + bc_ref[...]
    c0_ref[...] = jnp.broadcast_to(c0, c0_ref.shape)


def _tc2(P, nd_col, ns_col, b1r, W2, Wc, b2r, bcr):
    g = 8
    rb = NPAD // g
    return pl.pallas_call(
        _tc2_body,
        grid=(g,),
        in_specs=[
            pl.BlockSpec((2, rb, HF), lambda i: (0, i, 0)),
            pl.BlockSpec((rb, 1), lambda i: (i, 0)),
            pl.BlockSpec((rb, 1), lambda i: (i, 0)),
            pl.BlockSpec((1, F), lambda i: (0, 0)),
            pl.BlockSpec((F, F), lambda i: (0, 0)),
            pl.BlockSpec((F, 1), lambda i: (0, 0)),
            pl.BlockSpec((1, F), lambda i: (0, 0)),
            pl.BlockSpec((1, 1), lambda i: (0, 0)),
        ],
        out_specs=[
            pl.BlockSpec((rb, 1), lambda i: (i, 0)),
            pl.BlockSpec((1, F), lambda i: (0, 0)),
        ],
        out_shape=[
            jax.ShapeDtypeStruct((NPAD, 1), jnp.float32),
            jax.ShapeDtypeStruct((1, F), jnp.float32),
        ],
    )(P, nd_col, ns_col, b1r, W2, Wc, b2r, bcr)


def kernel(x, edge_index, W1, b1, W2, b2, Wc, bc):
    edges16 = edge_index.reshape(2, 16, EROWS // 16, B)
    src16 = edge_index[0].reshape(16, EROWS // 16, B)
    dst16 = edge_index[1].reshape(16, EROWS // 16, B)
    x_pad = jnp.pad(x, ((0, NPAD - N), (0, 0)))

    degs = _sc_deg_kernel()(edges16)
    dego = degs[0].reshape(NPAD, 1)
    degi = degs[1].reshape(NPAD, 1)

    zsplit, ns_col, nd_col = _tc1(x_pad, W1, dego, degi)
    zcat = zsplit.reshape(2 * NPAD, HF)

    P = _sc_agg_kernel()(src16, dst16, zcat)

    vn_col, c0 = _tc2(P, nd_col, ns_col, b1.reshape(1, F), W2, Wc,
                      b2.reshape(1, F), bc.reshape(1, 1))

    y_pad = _sc_final_kernel()(src16, dst16, vn_col.reshape(NPAD),
                               nd_col.reshape(NPAD), c0.reshape(F))
    return y_pad[:N].reshape(N, 1)


# speedup vs baseline: 17.6987x; 1.0237x over previous
import functools

import jax
import jax.numpy as jnp
from jax import lax
from jax.experimental import pallas as pl
from jax.experimental.pallas import tpu as pltpu
from jax.experimental.pallas import tpu_sc as plsc

N = 10000
E = 320000
F = 128
NPAD = 10240
RPT = NPAD // 16
B = 80
EROWS = E // B

_MESH = dict(core_axis_name="c", subcore_axis_name="s", num_cores=2,
             num_subcores=16)


def _zero_rows(ref, nrows, ncols):
    zeros = jnp.zeros((16,), jnp.float32)

    def body(i, carry):
        for j in range(ncols // 16):
            ref[i, pl.ds(j * 16, 16)] = zeros
        return carry

    lax.fori_loop(0, nrows, body, 0)


def _zero_vec(ref, n):
    zeros = jnp.zeros((16,), jnp.float32)

    def body(i, carry):
        ref[pl.ds(i * 16, 16)] = zeros
        return carry

    lax.fori_loop(0, n // 16, body, 0)


def _sc_deg_kernel():
    return pl.kernel(
        _sc_deg_body,
        out_type=jax.ShapeDtypeStruct((2, NPAD), jnp.float32),
        mesh=plsc.VectorSubcoreMesh(**_MESH),
        scratch_types=[
            pltpu.VMEM((EROWS // 16, B), jnp.int32),
            pltpu.VMEM((B,), jnp.float32),
            pltpu.VMEM((RPT,), jnp.float32),
            pltpu.VMEM_SHARED((NPAD,), jnp.float32),
            pltpu.SemaphoreType.DMA,
        ],
    )


def _sc_deg_body(edges_hbm, out_hbm, idxv, ones, zv, acc, sem):
    c = lax.axis_index("c")
    s = lax.axis_index("s")
    base = s * RPT
    nrows = EROWS // 16

    o = jnp.ones((16,), jnp.float32)
    for j in range(B // 16):
        ones[pl.ds(j * 16, 16)] = o
    _zero_vec(zv, RPT)
    pltpu.sync_copy(zv, acc.at[pl.ds(base, RPT)])
    pltpu.sync_copy(edges_hbm.at[c, s], idxv)
    plsc.subcore_barrier()

    def fire(i, carry):
        pltpu.async_copy(ones, acc.at[idxv.at[i]], sem, add=True)
        return carry

    lax.fori_loop(0, nrows, fire, 0)

    def drain(i, carry):
        pltpu.make_async_copy(ones, acc.at[idxv.at[i]], sem).wait()
        return carry

    lax.fori_loop(0, nrows, drain, 0)
    plsc.subcore_barrier()
    pltpu.sync_copy(acc.at[pl.ds(base, RPT)], out_hbm.at[c, pl.ds(base, RPT)])


HF = F // 2
RING = 8
LOOK = 4


def _sc_agg_body(src_hbm, dst_hbm, zcat_hbm, out_hbm, sidx, didx, rows0,
                 rows1, rows2, rows3, rows4, rows5, rows6, rows7, acc,
                 gs0, gs1, gs2, gs3, gs4, gs5, gs6, gs7,
                 ss0, ss1, ss2, ss3, ss4, ss5, ss6, ss7):
    c = lax.axis_index("c")
    s = lax.axis_index("s")
    base = s * RPT
    nb = EROWS // 16

    _zero_rows(rows0, B, HF)

    def zcopy(i, carry):
        pltpu.sync_copy(rows0, acc.at[pl.ds(base + i * B, B)])
        return carry

    lax.fori_loop(0, RPT // B, zcopy, 0)

    pltpu.sync_copy(src_hbm.at[s], sidx)
    pltpu.sync_copy(dst_hbm.at[s], didx)

    off = jnp.full((16,), c * NPAD, jnp.int32)

    def adj(i, carry):
        for j in range(B // 16):
            sidx[i, pl.ds(j * 16, 16)] = sidx[i, pl.ds(j * 16, 16)] + off
        return carry

    lax.fori_loop(0, nb, adj, 0)
    plsc.subcore_barrier()

    rows = (rows0, rows1, rows2, rows3, rows4, rows5, rows6, rows7)
    gsem = (gs0, gs1, gs2, gs3, gs4, gs5, gs6, gs7)
    ssem = (ss0, ss1, ss2, ss3, ss4, ss5, ss6, ss7)

    def gath(b, j):
        return pltpu.async_copy(zcat_hbm.at[sidx.at[b]], rows[j], gsem[j])

    def gath_w(b, j):
        pltpu.make_async_copy(zcat_hbm.at[sidx.at[b]], rows[j],
                              gsem[j]).wait()

    def scat(b, j):
        return pltpu.async_copy(rows[j], acc.at[didx.at[b]], ssem[j],
                                add=True)

    def scat_w(b, j):
        pltpu.make_async_copy(rows[j], acc.at[didx.at[b]], ssem[j]).wait()

    for k in range(LOOK):
        gath(k, k)

    def body(i, carry):
        for k in range(RING):
            m = RING * i + k

            @pl.when(m >= LOOK)
            def _():
                scat_w(m - LOOK, (k + RING - LOOK) % RING)

            gath(m + LOOK, (k + LOOK) % RING)
            gath_w(m, k)
            scat(m, k)
        return carry

    nsteady = (nb - LOOK) // RING
    lax.fori_loop(0, nsteady, body, 0)
    for m in range(RING * nsteady, nb):
        scat_w(m - LOOK, (m - LOOK) % RING)
        if m + LOOK < nb:
            gath(m + LOOK, (m + LOOK) % RING)
        gath_w(m, m % RING)
        scat(m, m % RING)
    for m in range(nb - LOOK, nb):
        scat_w(m, m % RING)

    plsc.subcore_barrier()
    pltpu.sync_copy(acc.at[pl.ds(base, RPT)],
                    out_hbm.at[c, pl.ds(base, RPT)])


def _sc_agg_kernel():
    return pl.kernel(
        _sc_agg_body,
        out_type=jax.ShapeDtypeStruct((2, NPAD, HF), jnp.float32),
        mesh=plsc.VectorSubcoreMesh(**_MESH),
        scratch_types=(
            [pltpu.VMEM((EROWS // 16, B), jnp.int32)] * 2
            + [pltpu.VMEM((B, HF), jnp.float32)] * RING
            + [pltpu.VMEM_SHARED((NPAD, HF), jnp.float32)]
            + [pltpu.SemaphoreType.DMA] * (2 * RING)
        ),
        compiler_params=pltpu.CompilerParams(use_tc_tiling_on_sc=False),
    )


def _sc_final_body(src_hbm, dst_hbm, vn_hbm, nd_hbm, c0_hbm, y_hbm, sidx,
                   didx, vals, vns, workv, ndv, c0v, acc, sem):
    c = lax.axis_index("c")
    s = lax.axis_index("s")
    base = s * RPT
    nrows = EROWS // 16

    @pl.when(c == 0)
    def _():
        _zero_vec(workv, RPT)
        pltpu.sync_copy(workv, acc.at[pl.ds(base, RPT)])
        pltpu.sync_copy(src_hbm.at[s], sidx)
        pltpu.sync_copy(dst_hbm.at[s], didx)
        pltpu.sync_copy(vn_hbm.at[pl.ds(base, RPT)],
                        vns.at[pl.ds(base, RPT)])
        plsc.subcore_barrier()

        def fire_g(i, carry):
            pltpu.async_copy(vns.at[sidx.at[i]], vals.at[i], sem)
            return carry

        lax.fori_loop(0, nrows, fire_g, 0)

        def drain_g(i, carry):
            pltpu.make_async_copy(vns.at[sidx.at[i]], vals.at[i],
                                  sem).wait()
            return carry

        lax.fori_loop(0, nrows, drain_g, 0)

        def fire_s(i, carry):
            pltpu.async_copy(vals.at[i], acc.at[didx.at[i]], sem, add=True)
            return carry

        lax.fori_loop(0, nrows, fire_s, 0)

        def drain_s(i, carry):
            pltpu.make_async_copy(vals.at[i], acc.at[didx.at[i]], sem).wait()
            return carry

        lax.fori_loop(0, nrows, drain_s, 0)
        plsc.subcore_barrier()
        pltpu.sync_copy(acc.at[pl.ds(base, RPT)], workv)
        pltpu.sync_copy(nd_hbm.at[pl.ds(base, RPT)], ndv)
        pltpu.sync_copy(c0_hbm.at[pl.ds(0, 16)], c0v)
        c016 = c0v[pl.ds(0, 16)]

        def fin(i, carry):
            a = workv[pl.ds(i * 16, 16)]
            d = ndv[pl.ds(i * 16, 16)]
            workv[pl.ds(i * 16, 16)] = a * d + c016
            return carry

        lax.fori_loop(0, RPT // 16, fin, 0)
        pltpu.sync_copy(workv, y_hbm.at[pl.ds(base, RPT)])


def _sc_final_kernel():
    return pl.kernel(
        _sc_final_body,
        out_type=jax.ShapeDtypeStruct((NPAD,), jnp.float32),
        mesh=plsc.VectorSubcoreMesh(**_MESH),
        scratch_types=[
            pltpu.VMEM((EROWS // 16, B), jnp.int32),
            pltpu.VMEM((EROWS // 16, B), jnp.int32),
            pltpu.VMEM((EROWS // 16, B), jnp.float32),
            pltpu.VMEM_SHARED((NPAD,), jnp.float32),
            pltpu.VMEM((RPT,), jnp.float32),
            pltpu.VMEM((RPT,), jnp.float32),
            pltpu.VMEM((16,), jnp.float32),
            pltpu.VMEM_SHARED((NPAD,), jnp.float32),
            pltpu.SemaphoreType.DMA,
        ],
    )


def _tc1_body(x_ref, w_ref, do_ref, di_ref, zn_ref, ns_ref, nd_ref):
    do = do_ref[...]
    di = di_ref[...]
    ns = jnp.where(do > 0, lax.rsqrt(jnp.maximum(do, 1.0)), 0.0)
    nd = jnp.where(di > 0, lax.rsqrt(jnp.maximum(di, 1.0)), 0.0)
    z = jnp.dot(x_ref[...], w_ref[0], preferred_element_type=jnp.float32)
    zn_ref[...] = (z * ns)[None]
    ns_ref[...] = ns
    nd_ref[...] = nd


def _tc1(x_pad, W1, dego, degi):
    g = 8
    rb = NPAD // g
    return pl.pallas_call(
        _tc1_body,
        grid=(2, g),
        in_specs=[
            pl.BlockSpec((rb, F), lambda j, i: (i, 0)),
            pl.BlockSpec((1, F, HF), lambda j, i: (j, 0, 0)),
            pl.BlockSpec((rb, 1), lambda j, i: (i, 0)),
            pl.BlockSpec((rb, 1), lambda j, i: (i, 0)),
        ],
        out_specs=[
            pl.BlockSpec((1, rb, HF), lambda j, i: (j, i, 0)),
            pl.BlockSpec((rb, 1), lambda j, i: (i, 0)),
            pl.BlockSpec((rb, 1), lambda j, i: (i, 0)),
        ],
        out_shape=[
            jax.ShapeDtypeStruct((2, NPAD, HF), jnp.float32),
            jax.ShapeDtypeStruct((NPAD, 1), jnp.float32),
            jax.ShapeDtypeStruct((NPAD, 1), jnp.float32),
        ],
    )(x_pad, jnp.stack([W1[:, :HF], W1[:, HF:]]), dego, degi)


def _tc2_body(p_ref, nd_ref, ns_ref, b1_ref, w2_ref, wc_ref, b2_ref, bc_ref,
              vn_ref, c0_ref):
    agg = jnp.concatenate([p_ref[0], p_ref[1]], axis=1)
    h1 = jnp.maximum(agg * nd_ref[...] + b1_ref[...], 0.0)
    w2c = jnp.dot(w2_ref[...], wc_ref[...], preferred_element_type=jnp.float32)
    v = jnp.dot(h1, w2c, preferred_element_type=jnp.float32)
    vn_ref[...] = v * ns_ref[...]
    c0 = jnp.dot(b2_ref[...], wc_ref[...],
                 preferred_element_type=jnp.float32) + bc_ref[...]
    c0_ref[...] = jnp.broadcast_to(c0, c0_ref.shape)


def _tc2(P, nd_col, ns_col, b1r, W2, Wc, b2r, bcr):
    g = 8
    rb = NPAD // g
    return pl.pallas_call(
        _tc2_body,
        grid=(g,),
        in_specs=[
            pl.BlockSpec((2, rb, HF), lambda i: (0, i, 0)),
            pl.BlockSpec((rb, 1), lambda i: (i, 0)),
            pl.BlockSpec((rb, 1), lambda i: (i, 0)),
            pl.BlockSpec((1, F), lambda i: (0, 0)),
            pl.BlockSpec((F, F), lambda i: (0, 0)),
            pl.BlockSpec((F, 1), lambda i: (0, 0)),
            pl.BlockSpec((1, F), lambda i: (0, 0)),
            pl.BlockSpec((1, 1), lambda i: (0, 0)),
        ],
        out_specs=[
            pl.BlockSpec((rb, 1), lambda i: (i, 0)),
            pl.BlockSpec((1, F), lambda i: (0, 0)),
        ],
        out_shape=[
            jax.ShapeDtypeStruct((NPAD, 1), jnp.float32),
            jax.ShapeDtypeStruct((1, F), jnp.float32),
        ],
    )(P, nd_col, ns_col, b1r, W2, Wc, b2r, bcr)


def kernel(x, edge_index, W1, b1, W2, b2, Wc, bc):
    edges16 = edge_index.reshape(2, 16, EROWS // 16, B)
    src16 = edge_index[0].reshape(16, EROWS // 16, B)
    dst16 = edge_index[1].reshape(16, EROWS // 16, B)
    x_pad = jnp.pad(x, ((0, NPAD - N), (0, 0)))

    degs = _sc_deg_kernel()(edges16)
    dego = degs[0].reshape(NPAD, 1)
    degi = degs[1].reshape(NPAD, 1)

    zsplit, ns_col, nd_col = _tc1(x_pad, W1, dego, degi)
    zcat = zsplit.reshape(2 * NPAD, HF)

    P = _sc_agg_kernel()(src16, dst16, zcat)

    vn_col, c0 = _tc2(P, nd_col, ns_col, b1.reshape(1, F), W2, Wc,
                      b2.reshape(1, F), bc.reshape(1, 1))

    y_pad = _sc_final_kernel()(src16, dst16, vn_col.reshape(NPAD),
                               nd_col.reshape(NPAD), c0.reshape(F))
    return y_pad[:N].reshape(N, 1)


# speedup vs baseline: 18.0062x; 1.0174x over previous
import functools

import jax
import jax.numpy as jnp
from jax import lax
from jax.experimental import pallas as pl
from jax.experimental.pallas import tpu as pltpu
from jax.experimental.pallas import tpu_sc as plsc

N = 10000
E = 320000
F = 128
NPAD = 10240
RPT = NPAD // 16
B = 80
EROWS = E // B

_MESH = dict(core_axis_name="c", subcore_axis_name="s", num_cores=2,
             num_subcores=16)


def _zero_rows(ref, nrows, ncols):
    zeros = jnp.zeros((16,), jnp.float32)

    def body(i, carry):
        for j in range(ncols // 16):
            ref[i, pl.ds(j * 16, 16)] = zeros
        return carry

    lax.fori_loop(0, nrows, body, 0)


def _zero_vec(ref, n):
    zeros = jnp.zeros((16,), jnp.float32)

    def body(i, carry):
        ref[pl.ds(i * 16, 16)] = zeros
        return carry

    lax.fori_loop(0, n // 16, body, 0)


def _sc_deg_kernel():
    return pl.kernel(
        _sc_deg_body,
        out_type=jax.ShapeDtypeStruct((2, NPAD), jnp.float32),
        mesh=plsc.VectorSubcoreMesh(**_MESH),
        scratch_types=[
            pltpu.VMEM((EROWS // 16, B), jnp.int32),
            pltpu.VMEM((B,), jnp.float32),
            pltpu.VMEM((RPT,), jnp.float32),
            pltpu.VMEM_SHARED((NPAD,), jnp.float32),
            pltpu.SemaphoreType.DMA,
        ],
    )


def _sc_deg_body(edges_hbm, out_hbm, idxv, ones, zv, acc, sem):
    c = lax.axis_index("c")
    s = lax.axis_index("s")
    base = s * RPT
    nrows = EROWS // 16

    o = jnp.ones((16,), jnp.float32)
    for j in range(B // 16):
        ones[pl.ds(j * 16, 16)] = o
    _zero_vec(zv, RPT)
    pltpu.sync_copy(zv, acc.at[pl.ds(base, RPT)])
    pltpu.sync_copy(edges_hbm.at[c, s], idxv)
    plsc.subcore_barrier()

    def fire(i, carry):
        pltpu.async_copy(ones, acc.at[idxv.at[i]], sem, add=True)
        return carry

    lax.fori_loop(0, nrows, fire, 0)

    def drain(i, carry):
        pltpu.make_async_copy(ones, acc.at[idxv.at[i]], sem).wait()
        return carry

    lax.fori_loop(0, nrows, drain, 0)
    plsc.subcore_barrier()
    pltpu.sync_copy(acc.at[pl.ds(base, RPT)], out_hbm.at[c, pl.ds(base, RPT)])


HF = F // 2
RING = 8
LOOK = 4


def _sc_agg_body(src_hbm, dst_hbm, zcat_hbm, out_hbm, sidx, didx, rows0,
                 rows1, rows2, rows3, rows4, rows5, rows6, rows7, acc,
                 gs0, gs1, gs2, gs3, gs4, gs5, gs6, gs7,
                 ss0, ss1, ss2, ss3, ss4, ss5, ss6, ss7):
    c = lax.axis_index("c")
    s = lax.axis_index("s")
    base = s * RPT
    nb = EROWS // 16

    _zero_rows(rows0, B, HF)

    def zcopy(i, carry):
        pltpu.sync_copy(rows0, acc.at[pl.ds(base + i * B, B)])
        return carry

    lax.fori_loop(0, RPT // B, zcopy, 0)

    pltpu.sync_copy(src_hbm.at[s], sidx)
    pltpu.sync_copy(dst_hbm.at[s], didx)

    off = jnp.full((16,), c * NPAD, jnp.int32)

    def adj(i, carry):
        for j in range(B // 16):
            sidx[i, pl.ds(j * 16, 16)] = sidx[i, pl.ds(j * 16, 16)] + off
        return carry

    lax.fori_loop(0, nb, adj, 0)
    plsc.subcore_barrier()

    rows = (rows0, rows1, rows2, rows3, rows4, rows5, rows6, rows7)
    gsem = (gs0, gs1, gs2, gs3, gs4, gs5, gs6, gs7)
    ssem = (ss0, ss1, ss2, ss3, ss4, ss5, ss6, ss7)

    def gath(b, j):
        return pltpu.async_copy(zcat_hbm.at[sidx.at[b]], rows[j], gsem[j])

    def gath_w(b, j):
        pltpu.make_async_copy(zcat_hbm.at[sidx.at[b]], rows[j],
                              gsem[j]).wait()

    def scat(b, j):
        return pltpu.async_copy(rows[j], acc.at[didx.at[b]], ssem[j],
                                add=True)

    def scat_w(b, j):
        pltpu.make_async_copy(rows[j], acc.at[didx.at[b]], ssem[j]).wait()

    for k in range(LOOK):
        gath(k, k)

    def body(i, carry):
        for k in range(RING):
            m = RING * i + k

            @pl.when(m >= LOOK)
            def _():
                scat_w(m - LOOK, (k + RING - LOOK) % RING)

            gath(m + LOOK, (k + LOOK) % RING)
            gath_w(m, k)
            scat(m, k)
        return carry

    nsteady = (nb - LOOK) // RING
    lax.fori_loop(0, nsteady, body, 0)
    for m in range(RING * nsteady, nb):
        scat_w(m - LOOK, (m - LOOK) % RING)
        if m + LOOK < nb:
            gath(m + LOOK, (m + LOOK) % RING)
        gath_w(m, m % RING)
        scat(m, m % RING)
    for m in range(nb - LOOK, nb):
        scat_w(m, m % RING)

    plsc.subcore_barrier()
    pltpu.sync_copy(acc.at[pl.ds(base, RPT)],
                    out_hbm.at[c, pl.ds(base, RPT)])


def _sc_agg_kernel():
    return pl.kernel(
        _sc_agg_body,
        out_type=jax.ShapeDtypeStruct((2, NPAD, HF), jnp.float32),
        mesh=plsc.VectorSubcoreMesh(**_MESH),
        scratch_types=(
            [pltpu.VMEM((EROWS // 16, B), jnp.int32)] * 2
            + [pltpu.VMEM((B, HF), jnp.float32)] * RING
            + [pltpu.VMEM_SHARED((NPAD, HF), jnp.float32)]
            + [pltpu.SemaphoreType.DMA] * (2 * RING)
        ),
        compiler_params=pltpu.CompilerParams(use_tc_tiling_on_sc=False),
    )


def _sc_final_body(src_hbm, dst_hbm, vn_hbm, nd_hbm, c0_hbm, y_hbm, sidx,
                   didx, vals, vns, workv, ndv, c0v, acc, sem):
    c = lax.axis_index("c")
    s = lax.axis_index("s")
    base = s * RPT
    nrows = EROWS // 16

    @pl.when(c == 0)
    def _():
        _zero_vec(workv, RPT)
        pltpu.sync_copy(workv, acc.at[pl.ds(base, RPT)])
        pltpu.sync_copy(src_hbm.at[s], sidx)
        pltpu.sync_copy(dst_hbm.at[s], didx)
        pltpu.sync_copy(vn_hbm.at[pl.ds(base, RPT)],
                        vns.at[pl.ds(base, RPT)])
        plsc.subcore_barrier()

        def fire_g(i, carry):
            pltpu.async_copy(vns.at[sidx.at[i]], vals.at[i], sem)
            return carry

        lax.fori_loop(0, nrows, fire_g, 0)

        def drain_g(i, carry):
            pltpu.make_async_copy(vns.at[sidx.at[i]], vals.at[i],
                                  sem).wait()
            return carry

        lax.fori_loop(0, nrows, drain_g, 0)

        def fire_s(i, carry):
            pltpu.async_copy(vals.at[i], acc.at[didx.at[i]], sem, add=True)
            return carry

        lax.fori_loop(0, nrows, fire_s, 0)

        def drain_s(i, carry):
            pltpu.make_async_copy(vals.at[i], acc.at[didx.at[i]], sem).wait()
            return carry

        lax.fori_loop(0, nrows, drain_s, 0)
        plsc.subcore_barrier()
        pltpu.sync_copy(acc.at[pl.ds(base, RPT)], workv)
        pltpu.sync_copy(nd_hbm.at[pl.ds(base, RPT)], ndv)
        pltpu.sync_copy(c0_hbm.at[pl.ds(0, 16)], c0v)
        c016 = c0v[pl.ds(0, 16)]

        def fin(i, carry):
            a = workv[pl.ds(i * 16, 16)]
            d = ndv[pl.ds(i * 16, 16)]
            workv[pl.ds(i * 16, 16)] = a * d + c016
            return carry

        lax.fori_loop(0, RPT // 16, fin, 0)
        pltpu.sync_copy(workv, y_hbm.at[pl.ds(base, RPT)])


def _sc_final_kernel():
    return pl.kernel(
        _sc_final_body,
        out_type=jax.ShapeDtypeStruct((NPAD,), jnp.float32),
        mesh=plsc.VectorSubcoreMesh(**_MESH),
        scratch_types=[
            pltpu.VMEM((EROWS // 16, B), jnp.int32),
            pltpu.VMEM((EROWS // 16, B), jnp.int32),
            pltpu.VMEM((EROWS // 16, B), jnp.float32),
            pltpu.VMEM_SHARED((NPAD,), jnp.float32),
            pltpu.VMEM((RPT,), jnp.float32),
            pltpu.VMEM((RPT,), jnp.float32),
            pltpu.VMEM((16,), jnp.float32),
            pltpu.VMEM_SHARED((NPAD,), jnp.float32),
            pltpu.SemaphoreType.DMA,
        ],
    )


def _tc1_body(x_ref, w_ref, do_ref, di_ref, zn_ref, ns_ref, nd_ref):
    do = do_ref[...]
    di = di_ref[...]
    ns = jnp.where(do > 0, lax.rsqrt(jnp.maximum(do, 1.0)), 0.0)
    nd = jnp.where(di > 0, lax.rsqrt(jnp.maximum(di, 1.0)), 0.0)
    z = jnp.dot(x_ref[...], w_ref[0], preferred_element_type=jnp.float32)
    zn_ref[...] = z * ns
    ns_ref[...] = ns
    nd_ref[...] = nd


def _tc1(x_pad, W1, dego, degi):
    g = 8
    rb = NPAD // g
    return pl.pallas_call(
        _tc1_body,
        grid=(8, 2),
        in_specs=[
            pl.BlockSpec((rb, F), lambda i, j: (i, 0)),
            pl.BlockSpec((1, F, HF), lambda i, j: (j, 0, 0)),
            pl.BlockSpec((rb, 1), lambda i, j: (i, 0)),
            pl.BlockSpec((rb, 1), lambda i, j: (i, 0)),
        ],
        out_specs=[
            pl.BlockSpec((rb, HF), lambda i, j: (j * g + i, 0)),
            pl.BlockSpec((rb, 1), lambda i, j: (i, 0)),
            pl.BlockSpec((rb, 1), lambda i, j: (i, 0)),
        ],
        out_shape=[
            jax.ShapeDtypeStruct((2 * NPAD, HF), jnp.float32),
            jax.ShapeDtypeStruct((NPAD, 1), jnp.float32),
            jax.ShapeDtypeStruct((NPAD, 1), jnp.float32),
        ],
    )(x_pad.astype(jnp.bfloat16),
      jnp.stack([W1[:, :HF], W1[:, HF:]]).astype(jnp.bfloat16), dego, degi)


def _tc2_body(p_ref, nd_ref, ns_ref, b1_ref, w2_ref, wc_ref, b2_ref, bc_ref,
              vn_ref, c0_ref):
    agg = jnp.concatenate([p_ref[0], p_ref[1]], axis=1)
    h1 = jnp.maximum(agg * nd_ref[...] + b1_ref[...], 0.0)
    w2c = jnp.dot(w2_ref[...], wc_ref[...], preferred_element_type=jnp.float32)
    v = jnp.dot(h1, w2c, preferred_element_type=jnp.float32)
    vn_ref[...] = v * ns_ref[...]
    c0 = jnp.dot(b2_ref[...], wc_ref[...],
                 preferred_element_type=jnp.float32) + bc_ref[...]
    c0_ref[...] = jnp.broadcast_to(c0, c0_ref.shape)


def _tc2(P, nd_col, ns_col, b1r, W2, Wc, b2r, bcr):
    g = 8
    rb = NPAD // g
    return pl.pallas_call(
        _tc2_body,
        grid=(g,),
        in_specs=[
            pl.BlockSpec((2, rb, HF), lambda i: (0, i, 0)),
            pl.BlockSpec((rb, 1), lambda i: (i, 0)),
            pl.BlockSpec((rb, 1), lambda i: (i, 0)),
            pl.BlockSpec((1, F), lambda i: (0, 0)),
            pl.BlockSpec((F, F), lambda i: (0, 0)),
            pl.BlockSpec((F, 1), lambda i: (0, 0)),
            pl.BlockSpec((1, F), lambda i: (0, 0)),
            pl.BlockSpec((1, 1), lambda i: (0, 0)),
        ],
        out_specs=[
            pl.BlockSpec((rb, 1), lambda i: (i, 0)),
            pl.BlockSpec((1, F), lambda i: (0, 0)),
        ],
        out_shape=[
            jax.ShapeDtypeStruct((NPAD, 1), jnp.float32),
            jax.ShapeDtypeStruct((1, F), jnp.float32),
        ],
    )(P, nd_col, ns_col, b1r, W2, Wc, b2r, bcr)


def kernel(x, edge_index, W1, b1, W2, b2, Wc, bc):
    edges16 = edge_index.reshape(2, 16, EROWS // 16, B)
    src16 = edge_index[0].reshape(16, EROWS // 16, B)
    dst16 = edge_index[1].reshape(16, EROWS // 16, B)
    x_pad = jnp.pad(x, ((0, NPAD - N), (0, 0)))

    degs = _sc_deg_kernel()(edges16)
    dego = degs[0].reshape(NPAD, 1)
    degi = degs[1].reshape(NPAD, 1)

    zcat, ns_col, nd_col = _tc1(x_pad, W1, dego, degi)

    P = _sc_agg_kernel()(src16, dst16, zcat)

    vn_col, c0 = _tc2(P, nd_col, ns_col, b1.reshape(1, F), W2, Wc,
                      b2.reshape(1, F), bc.reshape(1, 1))

    y_pad = _sc_final_kernel()(src16, dst16, vn_col.reshape(NPAD),
                               nd_col.reshape(NPAD), c0.reshape(F))
    return y_pad[:N].reshape(N, 1)
